# baseline (device time: 93755 ns/iter reference)
import jax
import jax.numpy as jnp
from jax import lax
from jax.experimental import pallas as pl
from jax.experimental.pallas import tpu as pltpu

N_DEV = 32


def kernel(x, w_mat):
    m_per, k = x.shape
    _, n = w_mat.shape
    n_per = n // N_DEV
    m_tot = m_per * N_DEV

    def body(x_ref, w_ref, out_ref, wbuf, tiles,
             copy_sems, send_sems, recv_sems):
        me = lax.axis_index("i")

        def w_chunk_copy(t, slot):
            j = (me + t) % N_DEV
            return pltpu.make_async_copy(
                w_ref.at[:, pl.ds(j * n_per, n_per)],
                wbuf.at[slot],
                copy_sems.at[slot],
            )

        w_chunk_copy(0, 0).start()

        bsem = pltpu.get_barrier_semaphore()
        for d in range(1, N_DEV):
            pl.semaphore_signal(
                bsem, inc=1,
                device_id=((me + d) % N_DEV,),
                device_id_type=pl.DeviceIdType.MESH,
            )
        pl.semaphore_wait(bsem, N_DEV - 1)

        rdmas = []
        for t in range(N_DEV):
            slot = t % 2
            if t + 1 < N_DEV:
                w_chunk_copy(t + 1, (t + 1) % 2).start()
            w_chunk_copy(t, slot).wait()
            tile = jnp.dot(
                x_ref[...], wbuf[slot],
                preferred_element_type=jnp.float32,
                precision=lax.Precision.DEFAULT,
            )
            if t == 0:
                out_ref[pl.ds(me * m_per, m_per), :] = tile
            else:
                tiles[t, :, :] = tile
                rdma = pltpu.make_async_remote_copy(
                    src_ref=tiles.at[t],
                    dst_ref=out_ref.at[pl.ds(me * m_per, m_per)],
                    send_sem=send_sems.at[t],
                    recv_sem=recv_sems.at[t],
                    device_id=((me + t) % N_DEV,),
                    device_id_type=pl.DeviceIdType.MESH,
                )
                rdma.start()
                rdmas.append(rdma)

        for rdma in rdmas:
            rdma.wait_recv()
        for rdma in rdmas:
            rdma.wait_send()

    return pl.pallas_call(
        body,
        out_shape=jax.ShapeDtypeStruct((m_tot, n_per), jnp.float32),
        in_specs=[
            pl.BlockSpec(memory_space=pltpu.VMEM),
            pl.BlockSpec(memory_space=pltpu.MemorySpace.HBM),
        ],
        out_specs=pl.BlockSpec(memory_space=pltpu.VMEM),
        scratch_shapes=[
            pltpu.VMEM((2, k, n_per), jnp.float32),
            pltpu.VMEM((N_DEV, m_per, n_per), jnp.float32),
            pltpu.SemaphoreType.DMA((2,)),
            pltpu.SemaphoreType.DMA((N_DEV,)),
            pltpu.SemaphoreType.DMA((N_DEV,)),
        ],
        compiler_params=pltpu.CompilerParams(collective_id=0),
    )(x, w_mat)


# device time: 66770 ns/iter; 1.4041x vs baseline; 1.4041x over previous
import jax
import jax.numpy as jnp
from jax import lax
from jax.experimental import pallas as pl
from jax.experimental.pallas import tpu as pltpu

N_DEV = 32
CH = 8
TPC = N_DEV // CH


def kernel(x, w_mat):
    m_per, k = x.shape
    _, n = w_mat.shape
    n_per = n // N_DEV
    m_tot = m_per * N_DEV
    w_ch = n // CH

    def body(x_ref, w_ref, out_ref, xbf, wbuf, mbuf, tiles, rbuf,
             copy_sems, send_sems, recv_sems):
        me = lax.axis_index("i")
        mdiv = me // TPC
        mmod = me % TPC

        def w_chunk_copy(c, slot):
            cc = (mdiv + c) % CH
            return pltpu.make_async_copy(
                w_ref.at[:, pl.ds(cc * w_ch, w_ch)],
                wbuf.at[slot],
                copy_sems.at[slot],
            )

        w_chunk_copy(0, 0).start()
        xbf[...] = x_ref[...].astype(jnp.bfloat16)

        bsem = pltpu.get_barrier_semaphore()
        for d in range(1, N_DEV):
            pl.semaphore_signal(
                bsem, inc=1,
                device_id=((me + d) % N_DEV,),
                device_id_type=pl.DeviceIdType.MESH,
            )
        pl.semaphore_wait(bsem, N_DEV - 1)

        rdmas = []
        for c in range(CH):
            slot = c % 2
            if c + 1 < CH:
                w_chunk_copy(c + 1, (c + 1) % 2).start()
            w_chunk_copy(c, slot).wait()
            mbuf[...] = jnp.dot(
                xbf[...], wbuf[slot].astype(jnp.bfloat16),
                preferred_element_type=jnp.float32,
            ).astype(jnp.bfloat16)
            cc = (mdiv + c) % CH
            for r in range(TPC):
                t = TPC * c + r
                rr = (mmod + r) % TPC
                j = TPC * cc + rr
                if t == 0:
                    rbuf[pl.ds(me * m_per, m_per), :] = \
                        mbuf[:, pl.ds(rr * n_per, n_per)]
                else:
                    tiles[t, :, :] = mbuf[:, pl.ds(rr * n_per, n_per)]
                    rdma = pltpu.make_async_remote_copy(
                        src_ref=tiles.at[t],
                        dst_ref=rbuf.at[pl.ds(me * m_per, m_per)],
                        send_sem=send_sems.at[t],
                        recv_sem=recv_sems.at[t],
                        device_id=(j,),
                        device_id_type=pl.DeviceIdType.MESH,
                    )
                    rdma.start()
                    rdmas.append(rdma)

        for rdma in rdmas:
            rdma.wait_recv()
        out_ref[...] = rbuf[...].astype(jnp.float32)
        for rdma in rdmas:
            rdma.wait_send()

    return pl.pallas_call(
        body,
        out_shape=jax.ShapeDtypeStruct((m_tot, n_per), jnp.float32),
        in_specs=[
            pl.BlockSpec(memory_space=pltpu.VMEM),
            pl.BlockSpec(memory_space=pltpu.MemorySpace.HBM),
        ],
        out_specs=pl.BlockSpec(memory_space=pltpu.VMEM),
        scratch_shapes=[
            pltpu.VMEM((m_per, k), jnp.bfloat16),
            pltpu.VMEM((2, k, w_ch), jnp.float32),
            pltpu.VMEM((m_per, w_ch), jnp.bfloat16),
            pltpu.VMEM((N_DEV, m_per, n_per), jnp.bfloat16),
            pltpu.VMEM((m_tot, n_per), jnp.bfloat16),
            pltpu.SemaphoreType.DMA((2,)),
            pltpu.SemaphoreType.DMA((N_DEV,)),
            pltpu.SemaphoreType.DMA((N_DEV,)),
        ],
        compiler_params=pltpu.CompilerParams(
            collective_id=0, vmem_limit_bytes=64 * 1024 * 1024),
    )(x, w_mat)
